# speedup vs baseline: 20.2806x; 20.2806x over previous
import functools

import jax
import jax.numpy as jnp
import numpy as np
from jax.experimental import pallas as pl

NC = 32
NCOMP = 9
W = 5 * NC
BE = 4096
BN = 1000


def _silu(x):
    return x * jax.nn.sigmoid(x)


def _mlp_body(rbf, cij, w0t, b0, w1t, b1, w2t, b2, act01, act2):
    h = _silu(jnp.dot(rbf[...], w0t[...]) + b0[...])
    h = _silu(jnp.dot(h, w1t[...]) + b1[...])
    h = _silu(jnp.dot(h, w2t[...]) + b2[...]) * cij[...]
    act01[...] = h[:, : 2 * NC]
    act2[...] = h[:, 2 * NC :]


def _mlp(rbf_p, cij_p, w0t, b0, w1t, b1, w2t, b2):
    ep = rbf_p.shape[0]
    full = lambda i: (0, 0)
    blk = lambda i: (i, 0)
    return pl.pallas_call(
        _mlp_body,
        grid=(ep // BE,),
        in_specs=[
            pl.BlockSpec((BE, 8), blk),
            pl.BlockSpec((BE, 1), blk),
            pl.BlockSpec((8, NC), full),
            pl.BlockSpec((1, NC), full),
            pl.BlockSpec((NC, 2 * NC), full),
            pl.BlockSpec((1, 2 * NC), full),
            pl.BlockSpec((2 * NC, 3 * NC), full),
            pl.BlockSpec((1, 3 * NC), full),
        ],
        out_specs=[
            pl.BlockSpec((BE, 2 * NC), blk),
            pl.BlockSpec((BE, NC), blk),
        ],
        out_shape=[
            jax.ShapeDtypeStruct((ep, 2 * NC), jnp.float32),
            jax.ShapeDtypeStruct((ep, NC), jnp.float32),
        ],
    )(rbf_p, cij_p, w0t, b0, w1t, b1, w2t, b2)


def _prep_body(x9, wt0t, wt1t, wt2t, t1, t2, xn9):
    xr = x9[...]
    nrm1 = sum(xr[:, k * NC : (k + 1) * NC] ** 2 for k in range(NCOMP)) + 1.0
    inv = 1.0 / nrm1
    x = [xr[:, k * NC : (k + 1) * NC] * inv for k in range(NCOMP)]
    xn9[...] = jnp.concatenate(x, 1)
    i_ = (x[0] + x[4] + x[8]) * (1.0 / 3.0)
    a01 = 0.5 * (x[1] - x[3]); a02 = 0.5 * (x[2] - x[6]); a12 = 0.5 * (x[5] - x[7])
    s00 = x[0] - i_; s01 = 0.5 * (x[1] + x[3]); s02 = 0.5 * (x[2] + x[6])
    s11 = x[4] - i_; s12 = 0.5 * (x[5] + x[7])
    w1t_ = wt1t[...]
    t1[...] = jnp.concatenate(
        [jnp.dot(i_, wt0t[...]), jnp.dot(a01, w1t_), jnp.dot(a02, w1t_),
         jnp.dot(a12, w1t_), jnp.zeros_like(i_)], 1)
    w2t_ = wt2t[...]
    t2[...] = jnp.concatenate(
        [jnp.dot(s00, w2t_), jnp.dot(s01, w2t_), jnp.dot(s02, w2t_),
         jnp.dot(s11, w2t_), jnp.dot(s12, w2t_)], 1)


def _prep(x9, wt0t, wt1t, wt2t):
    n = x9.shape[0]
    full = lambda i: (0, 0)
    blk = lambda i: (i, 0)
    return pl.pallas_call(
        _prep_body,
        grid=(n // BN,),
        in_specs=[
            pl.BlockSpec((BN, NCOMP * NC), blk),
            pl.BlockSpec((NC, NC), full),
            pl.BlockSpec((NC, NC), full),
            pl.BlockSpec((NC, NC), full),
        ],
        out_specs=[
            pl.BlockSpec((BN, W), blk),
            pl.BlockSpec((BN, W), blk),
            pl.BlockSpec((BN, NCOMP * NC), blk),
        ],
        out_shape=[
            jax.ShapeDtypeStruct((n, W), jnp.float32),
            jax.ShapeDtypeStruct((n, W), jnp.float32),
            jax.ShapeDtypeStruct((n, NCOMP * NC), jnp.float32),
        ],
    )(x9, wt0t, wt1t, wt2t)


def _comps(ta, tb):
    i_, a01, a02, a12 = (ta[:, k * NC : (k + 1) * NC] for k in range(4))
    s00, s01, s02, s11, s12 = (tb[:, k * NC : (k + 1) * NC] for k in range(5))
    s22 = -(s00 + s11)
    return [[i_ + s00, a01 + s01, a02 + s02],
            [-a01 + s01, i_ + s11, a12 + s12],
            [-a02 + s02, -a12 + s12, i_ + s22]]


def _tail_body(m1, m2, t1, t2, xn9, wt3t, wt4t, wt5t, o9):
    m = _comps(m1[...], m2[...])
    y = _comps(t1[...], t2[...])
    C = [[sum(m[r][k] * y[k][c] + y[r][k] * m[k][c] for k in range(3))
          for c in range(3)] for r in range(3)]
    inv = 1.0 / (sum(C[r][c] ** 2 for r in range(3) for c in range(3)) + 1.0)
    ci = (C[0][0] + C[1][1] + C[2][2]) * (1.0 / 3.0)
    ca01 = 0.5 * (C[0][1] - C[1][0]); ca02 = 0.5 * (C[0][2] - C[2][0])
    ca12 = 0.5 * (C[1][2] - C[2][1])
    cs00 = C[0][0] - ci; cs01 = 0.5 * (C[0][1] + C[1][0])
    cs02 = 0.5 * (C[0][2] + C[2][0]); cs11 = C[1][1] - ci
    cs12 = 0.5 * (C[1][2] + C[2][1])
    w3, w4, w5 = wt3t[...], wt4t[...], wt5t[...]
    di = jnp.dot(ci * inv, w3)
    da01 = jnp.dot(ca01 * inv, w4); da02 = jnp.dot(ca02 * inv, w4)
    da12 = jnp.dot(ca12 * inv, w4)
    ds00 = jnp.dot(cs00 * inv, w5); ds01 = jnp.dot(cs01 * inv, w5)
    ds02 = jnp.dot(cs02 * inv, w5); ds11 = jnp.dot(cs11 * inv, w5)
    ds12 = jnp.dot(cs12 * inv, w5)
    ds22 = -(ds00 + ds11)
    d = [[di + ds00, da01 + ds01, da02 + ds02],
         [-da01 + ds01, di + ds11, da12 + ds12],
         [-da02 + ds02, -da12 + ds12, di + ds22]]
    xr = xn9[...]
    out = []
    for r in range(3):
        for c in range(3):
            dd = sum(d[r][k] * d[k][c] for k in range(3))
            out.append(xr[:, (3 * r + c) * NC : (3 * r + c + 1) * NC]
                       + d[r][c] + dd)
    o9[...] = jnp.concatenate(out, 1)


def _tail(m1p, m2p, t1p, t2p, xn9, wt3t, wt4t, wt5t):
    n = xn9.shape[0]
    full = lambda i: (0, 0)
    blk = lambda i: (i, 0)
    return pl.pallas_call(
        _tail_body,
        grid=(n // BN,),
        in_specs=[
            pl.BlockSpec((BN, W), blk),
            pl.BlockSpec((BN, W), blk),
            pl.BlockSpec((BN, W), blk),
            pl.BlockSpec((BN, W), blk),
            pl.BlockSpec((BN, NCOMP * NC), blk),
            pl.BlockSpec((NC, NC), full),
            pl.BlockSpec((NC, NC), full),
            pl.BlockSpec((NC, NC), full),
        ],
        out_specs=[pl.BlockSpec((BN, NCOMP * NC), blk)],
        out_shape=[jax.ShapeDtypeStruct((n, NCOMP * NC), jnp.float32)],
    )(m1p, m2p, t1p, t2p, xn9, wt3t, wt4t, wt5t)


def _messages(t1, t2, act01, act2, idx_i_p, idx_j_p, n):
    act0, act1 = act01[:, :NC], act01[:, NC:]
    w_row1 = jnp.concatenate([act0, act1, act1, act1, act0], 1)
    w_row2 = jnp.concatenate([act2] * 5, 1)
    m1 = jax.ops.segment_sum(w_row1 * t1[idx_j_p], idx_i_p, num_segments=n)
    m2 = jax.ops.segment_sum(w_row2 * t2[idx_j_p], idx_i_p, num_segments=n)
    return m1, m2


_PERM = np.array([3 * (p % NC) + p // NC for p in range(3 * NC)])


def kernel(X, cij, rbf, idx_i, idx_j, w0, b0, w1, b1, w2, b2,
           wt0, wt1, wt2, wt3, wt4, wt5):
    n = X.shape[0]
    e = rbf.shape[0]
    e_pad = ((e + 4095) // 4096) * 4096

    x9 = X.reshape(n, NC, NCOMP).swapaxes(1, 2).reshape(n, NCOMP * NC)
    rbf_p = jnp.zeros((e_pad, 8), jnp.float32).at[:e].set(rbf)
    cij_p = jnp.zeros((e_pad, 1), jnp.float32).at[:e, 0].set(cij)
    idx_i_p = jnp.zeros((e_pad,), jnp.int32).at[:e].set(idx_i)
    idx_j_p = jnp.zeros((e_pad,), jnp.int32).at[:e].set(idx_j)

    w2p = w2[_PERM]
    b2p = b2[_PERM]
    act01, act2 = _mlp(rbf_p, cij_p, w0.T, b0[None, :], w1.T, b1[None, :],
                       w2p.T, b2p[None, :])
    t1, t2, xn9 = _prep(x9, wt0.T, wt1.T, wt2.T)
    m1, m2 = _messages(t1, t2, act01, act2, idx_i_p, idx_j_p, n)
    o9 = _tail(m1, m2, t1, t2, xn9, wt3.T, wt4.T, wt5.T)[0]
    return o9.reshape(n, NCOMP, NC).swapaxes(1, 2).reshape(n, NC, 3, 3)


# speedup vs baseline: 50.8713x; 2.5084x over previous
import functools

import jax
import jax.numpy as jnp
import numpy as np
from jax.experimental import pallas as pl
from jax.experimental.pallas import tpu as pltpu
from jax.experimental.pallas import tpu_sc as plsc

NC = 32
NCOMP = 9
GW = 3 * NC
BE = 4096
BN = 1000


def _silu(x):
    return x * jax.nn.sigmoid(x)


def _mlp_body(rbf, cij, w0t, b0, w1t, b1, w2t, b2, act):
    h = _silu(jnp.dot(rbf[...], w0t[...]) + b0[...])
    h = _silu(jnp.dot(h, w1t[...]) + b1[...])
    act[...] = _silu(jnp.dot(h, w2t[...]) + b2[...]) * cij[...]


def _mlp(rbf_p, cij_p, w0t, b0, w1t, b1, w2t, b2):
    ep = rbf_p.shape[0]
    full = lambda i: (0, 0)
    blk = lambda i: (i, 0)
    return pl.pallas_call(
        _mlp_body,
        grid=(ep // BE,),
        in_specs=[
            pl.BlockSpec((BE, 8), blk),
            pl.BlockSpec((BE, 1), blk),
            pl.BlockSpec((8, NC), full),
            pl.BlockSpec((1, NC), full),
            pl.BlockSpec((NC, 2 * NC), full),
            pl.BlockSpec((1, 2 * NC), full),
            pl.BlockSpec((2 * NC, 3 * NC), full),
            pl.BlockSpec((1, 3 * NC), full),
        ],
        out_specs=[pl.BlockSpec((BE, 3 * NC), blk)],
        out_shape=[jax.ShapeDtypeStruct((ep, 3 * NC), jnp.float32)],
    )(rbf_p, cij_p, w0t, b0, w1t, b1, w2t, b2)[0]


def _prep_body(x9, wt0t, wt1t, wt2t, tg0, tg1, tg2, xn9):
    xr = x9[...]
    nrm1 = sum(xr[:, k * NC : (k + 1) * NC] ** 2 for k in range(NCOMP)) + 1.0
    inv = 1.0 / nrm1
    x = [xr[:, k * NC : (k + 1) * NC] * inv for k in range(NCOMP)]
    xn9[...] = jnp.concatenate(x, 1)
    i_ = (x[0] + x[4] + x[8]) * (1.0 / 3.0)
    a01 = 0.5 * (x[1] - x[3]); a02 = 0.5 * (x[2] - x[6]); a12 = 0.5 * (x[5] - x[7])
    s00 = x[0] - i_; s01 = 0.5 * (x[1] + x[3]); s02 = 0.5 * (x[2] + x[6])
    s11 = x[4] - i_; s12 = 0.5 * (x[5] + x[7])
    w1t_ = wt1t[...]
    w2t_ = wt2t[...]
    tg0[...] = jnp.concatenate(
        [jnp.dot(i_, wt0t[...]), jnp.dot(a01, w1t_), jnp.dot(a02, w1t_)], 1)
    tg1[...] = jnp.concatenate(
        [jnp.dot(a12, w1t_), jnp.dot(s00, w2t_), jnp.dot(s01, w2t_)], 1)
    tg2[...] = jnp.concatenate(
        [jnp.dot(s02, w2t_), jnp.dot(s11, w2t_), jnp.dot(s12, w2t_)], 1)


def _prep(x9, wt0t, wt1t, wt2t):
    n = x9.shape[0]
    full = lambda i: (0, 0)
    blk = lambda i: (i, 0)
    return pl.pallas_call(
        _prep_body,
        grid=(n // BN,),
        in_specs=[
            pl.BlockSpec((BN, NCOMP * NC), blk),
            pl.BlockSpec((NC, NC), full),
            pl.BlockSpec((NC, NC), full),
            pl.BlockSpec((NC, NC), full),
        ],
        out_specs=[
            pl.BlockSpec((BN, GW), blk),
            pl.BlockSpec((BN, GW), blk),
            pl.BlockSpec((BN, GW), blk),
            pl.BlockSpec((BN, NCOMP * NC), blk),
        ],
        out_shape=[
            jax.ShapeDtypeStruct((n, GW), jnp.float32),
            jax.ShapeDtypeStruct((n, GW), jnp.float32),
            jax.ShapeDtypeStruct((n, GW), jnp.float32),
            jax.ShapeDtypeStruct((n, NCOMP * NC), jnp.float32),
        ],
    )(x9, wt0t, wt1t, wt2t)


def _comps9(g0, g1, g2):
    i_, a01, a02 = (g0[:, k * NC : (k + 1) * NC] for k in range(3))
    a12, s00, s01 = (g1[:, k * NC : (k + 1) * NC] for k in range(3))
    s02, s11, s12 = (g2[:, k * NC : (k + 1) * NC] for k in range(3))
    s22 = -(s00 + s11)
    return [[i_ + s00, a01 + s01, a02 + s02],
            [-a01 + s01, i_ + s11, a12 + s12],
            [-a02 + s02, -a12 + s12, i_ + s22]]


def _tail_body(mg0, mg1, mg2a, mg2b, tg0, tg1, tg2, xn9, wt3t, wt4t, wt5t, o9):
    m = _comps9(mg0[...], mg1[...], mg2a[...] + mg2b[...])
    y = _comps9(tg0[...], tg1[...], tg2[...])
    C = [[sum(m[r][k] * y[k][c] + y[r][k] * m[k][c] for k in range(3))
          for c in range(3)] for r in range(3)]
    inv = 1.0 / (sum(C[r][c] ** 2 for r in range(3) for c in range(3)) + 1.0)
    ci = (C[0][0] + C[1][1] + C[2][2]) * (1.0 / 3.0)
    ca01 = 0.5 * (C[0][1] - C[1][0]); ca02 = 0.5 * (C[0][2] - C[2][0])
    ca12 = 0.5 * (C[1][2] - C[2][1])
    cs00 = C[0][0] - ci; cs01 = 0.5 * (C[0][1] + C[1][0])
    cs02 = 0.5 * (C[0][2] + C[2][0]); cs11 = C[1][1] - ci
    cs12 = 0.5 * (C[1][2] + C[2][1])
    w3, w4, w5 = wt3t[...], wt4t[...], wt5t[...]
    di = jnp.dot(ci * inv, w3)
    da01 = jnp.dot(ca01 * inv, w4); da02 = jnp.dot(ca02 * inv, w4)
    da12 = jnp.dot(ca12 * inv, w4)
    ds00 = jnp.dot(cs00 * inv, w5); ds01 = jnp.dot(cs01 * inv, w5)
    ds02 = jnp.dot(cs02 * inv, w5); ds11 = jnp.dot(cs11 * inv, w5)
    ds12 = jnp.dot(cs12 * inv, w5)
    ds22 = -(ds00 + ds11)
    d = [[di + ds00, da01 + ds01, da02 + ds02],
         [-da01 + ds01, di + ds11, da12 + ds12],
         [-da02 + ds02, -da12 + ds12, di + ds22]]
    xr = xn9[...]
    out = []
    for r in range(3):
        for c in range(3):
            dd = sum(d[r][k] * d[k][c] for k in range(3))
            out.append(xr[:, (3 * r + c) * NC : (3 * r + c + 1) * NC]
                       + d[r][c] + dd)
    o9[...] = jnp.concatenate(out, 1)


def _tail(mg0, mg1, mg2a, mg2b, tg0, tg1, tg2, xn9, wt3t, wt4t, wt5t):
    n = xn9.shape[0]
    full = lambda i: (0, 0)
    blk = lambda i: (i, 0)
    return pl.pallas_call(
        _tail_body,
        grid=(n // BN,),
        in_specs=[pl.BlockSpec((BN, GW), blk)] * 7
        + [pl.BlockSpec((BN, NCOMP * NC), blk)]
        + [pl.BlockSpec((NC, NC), full)] * 3,
        out_specs=[pl.BlockSpec((BN, NCOMP * NC), blk)],
        out_shape=[jax.ShapeDtypeStruct((n, NCOMP * NC), jnp.float32)],
    )(mg0, mg1, mg2a, mg2b, tg0, tg1, tg2, xn9, wt3t, wt4t, wt5t)[0]


NT = 16
CE = 128
_WMAP_A = [0, 1, 2, 3, 2, 3]
_WMAP_B = [0, 1, 0, 1, 0, 1]


def _sc_messages(tg0, tg1, tg2, act, idxi3, idxj3, n):
    ch = idxi3.shape[1]
    na = ((n + NT * CE - 1) // (NT * CE)) * (NT * CE)
    rpt = na // NT
    nzc = rpt // CE

    mesh = plsc.VectorSubcoreMesh(core_axis_name="c", subcore_axis_name="s")
    mshape = jax.ShapeDtypeStruct((na, GW), jnp.float32)

    @functools.partial(
        pl.kernel, mesh=mesh,
        compiler_params=pltpu.CompilerParams(use_tc_tiling_on_sc=False),
        out_type=[mshape, mshape, mshape, mshape],
        scratch_types=[
            pltpu.VMEM((ch, CE), jnp.int32),
            pltpu.VMEM((ch, CE), jnp.int32),
            pltpu.VMEM((CE, 4 * 16), jnp.float32),
            pltpu.VMEM((CE, 2 * 16), jnp.float32),
            pltpu.VMEM((CE, GW), jnp.float32),
            pltpu.VMEM_SHARED((na, GW), jnp.float32),
            pltpu.SemaphoreType.DMA,
        ],
    )
    def sc_fn(tg0_h, tg1_h, tg2_h, act_h, idxi_h, idxj_h,
              mg0_h, mg1_h, mg2a_h, mg2b_h,
              idxj_v, idxi_v, acta_v, actb_v, rows_v, acc, sem):
        cid = jax.lax.axis_index("c")
        sid = jax.lax.axis_index("s")
        tile_base = sid * (ch * CE)

        pltpu.sync_copy(idxj_h.at[sid], idxj_v)
        pltpu.sync_copy(idxi_h.at[sid], idxi_v)

        def zero_own_rows():
            def zb(r, carry):
                for t in range(GW // 16):
                    rows_v[r, pl.ds(16 * t, 16)] = jnp.zeros((16,), jnp.float32)
                return carry
            jax.lax.fori_loop(0, CE, zb, 0)
            for q in range(nzc):
                pltpu.sync_copy(rows_v, acc.at[pl.ds(sid * rpt + q * CE, CE)])

        def do_pass(tab_h, col0, act_v, nw, wmap, out_h, c0, c1):
            def chunk(ci, carry):
                pltpu.async_copy(tab_h.at[idxj_v.at[ci]], rows_v, sem).wait()
                pltpu.sync_copy(
                    act_h.at[pl.ds(tile_base + ci * CE, CE),
                             pl.ds(col0, 16 * nw)], act_v)

                def edge(e, c2):
                    wv = [act_v[e, pl.ds(16 * t, 16)] for t in range(nw)]
                    for t in range(GW // 16):
                        sl = pl.ds(16 * t, 16)
                        rows_v[e, sl] = rows_v[e, sl] * wv[wmap[t]]
                    return c2
                jax.lax.fori_loop(0, CE, edge, 0)
                pltpu.sync_copy(rows_v, acc.at[idxi_v.at[ci]], add=True)
                return carry
            jax.lax.fori_loop(c0, c1, chunk, 0)
            plsc.subcore_barrier()
            for q in range(nzc):
                rr = pl.ds(sid * rpt + q * CE, CE)
                pltpu.sync_copy(acc.at[rr], out_h.at[rr])

        zero_own_rows()
        plsc.subcore_barrier()

        @pl.when(cid == 0)
        def _():
            do_pass(tg0_h, 0, acta_v, 4, _WMAP_A, mg0_h, 0, ch)

        @pl.when(cid == 1)
        def _():
            do_pass(tg1_h, NC, acta_v, 4, _WMAP_A, mg1_h, 0, ch)

        zero_own_rows()
        plsc.subcore_barrier()

        @pl.when(cid == 0)
        def _():
            do_pass(tg2_h, 2 * NC, actb_v, 2, _WMAP_B, mg2a_h, 0, ch // 2)

        @pl.when(cid == 1)
        def _():
            do_pass(tg2_h, 2 * NC, actb_v, 2, _WMAP_B, mg2b_h, ch // 2, ch)

    mg0, mg1, mg2a, mg2b = sc_fn(tg0, tg1, tg2, act, idxi3, idxj3)
    return mg0[:n], mg1[:n], mg2a[:n], mg2b[:n]


_PERM = np.array([3 * (p % NC) + p // NC for p in range(3 * NC)])


def kernel(X, cij, rbf, idx_i, idx_j, w0, b0, w1, b1, w2, b2,
           wt0, wt1, wt2, wt3, wt4, wt5):
    n = X.shape[0]
    e = rbf.shape[0]
    e_pad = ((e + 4095) // 4096) * 4096

    x9 = X.reshape(n, NC, NCOMP).swapaxes(1, 2).reshape(n, NCOMP * NC)
    rbf_p = jnp.zeros((e_pad, 8), jnp.float32).at[:e].set(rbf)
    cij_p = jnp.zeros((e_pad, 1), jnp.float32).at[:e, 0].set(cij)
    ch = e_pad // (NT * CE)
    idxi3 = jnp.zeros((e_pad,), jnp.int32).at[:e].set(idx_i).reshape(NT, ch, CE)
    idxj3 = jnp.zeros((e_pad,), jnp.int32).at[:e].set(idx_j).reshape(NT, ch, CE)

    w2p = w2[_PERM]
    b2p = b2[_PERM]
    act = _mlp(rbf_p, cij_p, w0.T, b0[None, :], w1.T, b1[None, :],
               w2p.T, b2p[None, :])
    tg0, tg1, tg2, xn9 = _prep(x9, wt0.T, wt1.T, wt2.T)
    mg0, mg1, mg2a, mg2b = _sc_messages(tg0, tg1, tg2, act, idxi3, idxj3, n)
    o9 = _tail(mg0, mg1, mg2a, mg2b, tg0, tg1, tg2, xn9,
               wt3.T, wt4.T, wt5.T)
    return o9.reshape(n, NCOMP, NC).swapaxes(1, 2).reshape(n, NC, 3, 3)


# speedup vs baseline: 80.4133x; 1.5807x over previous
import functools

import jax
import jax.numpy as jnp
import numpy as np
from jax.experimental import pallas as pl
from jax.experimental.pallas import tpu as pltpu
from jax.experimental.pallas import tpu_sc as plsc

NC = 32
NCOMP = 9
GW = 3 * NC
BE = 2000
BN = 1000


def _silu(x):
    return x * jax.nn.sigmoid(x)


def _mlp_body(rbf, cij, w0t, b0, w1t, b1, w2t, b2, act):
    h = _silu(jnp.dot(rbf[...], w0t[...]) + b0[...])
    h = _silu(jnp.dot(h, w1t[...]) + b1[...])
    act[...] = _silu(jnp.dot(h, w2t[...]) + b2[...]) * cij[...]


def _mlp(rbf, cij2, w0t, b0, w1t, b1, w2t, b2):
    ep = rbf.shape[0]
    full = lambda i: (0, 0)
    blk = lambda i: (i, 0)
    return pl.pallas_call(
        _mlp_body,
        grid=(ep // BE,),
        in_specs=[
            pl.BlockSpec((BE, 8), blk),
            pl.BlockSpec((BE, 1), blk),
            pl.BlockSpec((8, NC), full),
            pl.BlockSpec((1, NC), full),
            pl.BlockSpec((NC, 2 * NC), full),
            pl.BlockSpec((1, 2 * NC), full),
            pl.BlockSpec((2 * NC, 3 * NC), full),
            pl.BlockSpec((1, 3 * NC), full),
        ],
        out_specs=[pl.BlockSpec((BE, 3 * NC), blk)],
        out_shape=[jax.ShapeDtypeStruct((ep, 3 * NC), jnp.float32)],
    )(rbf, cij2, w0t, b0, w1t, b1, w2t, b2)[0]


def _prep_body(xc, p, wt0t, wt1t, wt2t, tg0, tg1, tg2, xn9):
    x9r = jnp.dot(xc[...], p[...])
    nrm1 = sum(x9r[:, k * NC : (k + 1) * NC] ** 2 for k in range(NCOMP)) + 1.0
    inv = 1.0 / nrm1
    x = [x9r[:, k * NC : (k + 1) * NC] * inv for k in range(NCOMP)]
    xn9[...] = jnp.concatenate(x, 1)
    i_ = (x[0] + x[4] + x[8]) * (1.0 / 3.0)
    a01 = 0.5 * (x[1] - x[3]); a02 = 0.5 * (x[2] - x[6]); a12 = 0.5 * (x[5] - x[7])
    s00 = x[0] - i_; s01 = 0.5 * (x[1] + x[3]); s02 = 0.5 * (x[2] + x[6])
    s11 = x[4] - i_; s12 = 0.5 * (x[5] + x[7])
    w1t_ = wt1t[...]
    w2t_ = wt2t[...]
    tg0[...] = jnp.concatenate(
        [jnp.dot(i_, wt0t[...]), jnp.dot(a01, w1t_), jnp.dot(a02, w1t_)], 1)
    tg1[...] = jnp.concatenate(
        [jnp.dot(a12, w1t_), jnp.dot(s00, w2t_), jnp.dot(s01, w2t_)], 1)
    tg2[...] = jnp.concatenate(
        [jnp.dot(s02, w2t_), jnp.dot(s11, w2t_), jnp.dot(s12, w2t_)], 1)


def _prep(xc, p, wt0t, wt1t, wt2t):
    n = xc.shape[0]
    full = lambda i: (0, 0)
    blk = lambda i: (i, 0)
    return pl.pallas_call(
        _prep_body,
        grid=(n // BN,),
        in_specs=[
            pl.BlockSpec((BN, NCOMP * NC), blk),
            pl.BlockSpec((NCOMP * NC, NCOMP * NC), full),
            pl.BlockSpec((NC, NC), full),
            pl.BlockSpec((NC, NC), full),
            pl.BlockSpec((NC, NC), full),
        ],
        out_specs=[
            pl.BlockSpec((BN, GW), blk),
            pl.BlockSpec((BN, GW), blk),
            pl.BlockSpec((BN, GW), blk),
            pl.BlockSpec((BN, NCOMP * NC), blk),
        ],
        out_shape=[
            jax.ShapeDtypeStruct((n, GW), jnp.float32),
            jax.ShapeDtypeStruct((n, GW), jnp.float32),
            jax.ShapeDtypeStruct((n, GW), jnp.float32),
            jax.ShapeDtypeStruct((n, NCOMP * NC), jnp.float32),
        ],
    )(xc, p, wt0t, wt1t, wt2t)


def _comps9(g0, g1, g2):
    i_, a01, a02 = (g0[:, k * NC : (k + 1) * NC] for k in range(3))
    a12, s00, s01 = (g1[:, k * NC : (k + 1) * NC] for k in range(3))
    s02, s11, s12 = (g2[:, k * NC : (k + 1) * NC] for k in range(3))
    s22 = -(s00 + s11)
    return [[i_ + s00, a01 + s01, a02 + s02],
            [-a01 + s01, i_ + s11, a12 + s12],
            [-a02 + s02, -a12 + s12, i_ + s22]]


def _tail_body(mg0, mg1, mg2a, mg2b, tg0, tg1, tg2, xn9, pt, wt3t, wt4t, wt5t,
               oc):
    m = _comps9(mg0[...], mg1[...], mg2a[...] + mg2b[...])
    y = _comps9(tg0[...], tg1[...], tg2[...])
    C = [[sum(m[r][k] * y[k][c] + y[r][k] * m[k][c] for k in range(3))
          for c in range(3)] for r in range(3)]
    inv = 1.0 / (sum(C[r][c] ** 2 for r in range(3) for c in range(3)) + 1.0)
    ci = (C[0][0] + C[1][1] + C[2][2]) * (1.0 / 3.0)
    ca01 = 0.5 * (C[0][1] - C[1][0]); ca02 = 0.5 * (C[0][2] - C[2][0])
    ca12 = 0.5 * (C[1][2] - C[2][1])
    cs00 = C[0][0] - ci; cs01 = 0.5 * (C[0][1] + C[1][0])
    cs02 = 0.5 * (C[0][2] + C[2][0]); cs11 = C[1][1] - ci
    cs12 = 0.5 * (C[1][2] + C[2][1])
    w3, w4, w5 = wt3t[...], wt4t[...], wt5t[...]
    di = jnp.dot(ci * inv, w3)
    da01 = jnp.dot(ca01 * inv, w4); da02 = jnp.dot(ca02 * inv, w4)
    da12 = jnp.dot(ca12 * inv, w4)
    ds00 = jnp.dot(cs00 * inv, w5); ds01 = jnp.dot(cs01 * inv, w5)
    ds02 = jnp.dot(cs02 * inv, w5); ds11 = jnp.dot(cs11 * inv, w5)
    ds12 = jnp.dot(cs12 * inv, w5)
    ds22 = -(ds00 + ds11)
    d = [[di + ds00, da01 + ds01, da02 + ds02],
         [-da01 + ds01, di + ds11, da12 + ds12],
         [-da02 + ds02, -da12 + ds12, di + ds22]]
    xr = xn9[...]
    out = []
    for r in range(3):
        for c in range(3):
            dd = sum(d[r][k] * d[k][c] for k in range(3))
            out.append(xr[:, (3 * r + c) * NC : (3 * r + c + 1) * NC]
                       + d[r][c] + dd)
    oc[...] = jnp.dot(jnp.concatenate(out, 1), pt[...])


def _tail(mg0, mg1, mg2a, mg2b, tg0, tg1, tg2, xn9, pt, wt3t, wt4t, wt5t, n):
    full = lambda i: (0, 0)
    blk = lambda i: (i, 0)
    return pl.pallas_call(
        _tail_body,
        grid=(n // BN,),
        in_specs=[pl.BlockSpec((BN, GW), blk)] * 7
        + [pl.BlockSpec((BN, NCOMP * NC), blk)]
        + [pl.BlockSpec((NCOMP * NC, NCOMP * NC), full)]
        + [pl.BlockSpec((NC, NC), full)] * 3,
        out_specs=[pl.BlockSpec((BN, NCOMP * NC), blk)],
        out_shape=[jax.ShapeDtypeStruct((n, NCOMP * NC), jnp.float32)],
    )(mg0, mg1, mg2a, mg2b, tg0, tg1, tg2, xn9, pt, wt3t, wt4t, wt5t)[0]


NT = 16
_WMAP_A = [0, 1, 2, 3, 2, 3]
_WMAP_B = [0, 1, 0, 1, 0, 1]


def _sc_messages(tg0, tg1, tg2, act, idxi3, idxj3, n):
    ch = idxi3.shape[1]
    ce = idxi3.shape[2]
    na = ((n + NT * 128 - 1) // (NT * 128)) * (NT * 128)
    rpt = na // NT
    nzc = rpt // 128

    mesh = plsc.VectorSubcoreMesh(core_axis_name="c", subcore_axis_name="s")
    mshape = jax.ShapeDtypeStruct((na, GW), jnp.float32)

    @functools.partial(
        pl.kernel, mesh=mesh,
        compiler_params=pltpu.CompilerParams(use_tc_tiling_on_sc=False),
        out_type=[mshape, mshape, mshape, mshape],
        scratch_types=[
            pltpu.VMEM((ch, ce), jnp.int32),
            pltpu.VMEM((ch, ce), jnp.int32),
            pltpu.VMEM((2, ce, 4 * 16), jnp.float32),
            pltpu.VMEM((2, ce, 2 * 16), jnp.float32),
            pltpu.VMEM((2, ce, GW), jnp.float32),
            pltpu.VMEM((128, GW), jnp.float32),
            pltpu.VMEM_SHARED((na, GW), jnp.float32),
            pltpu.SemaphoreType.DMA,
            pltpu.SemaphoreType.DMA,
            pltpu.SemaphoreType.DMA,
            pltpu.SemaphoreType.DMA,
        ],
    )
    def sc_fn(tg0_h, tg1_h, tg2_h, act_h, idxi_h, idxj_h,
              mg0_h, mg1_h, mg2a_h, mg2b_h,
              idxj_v, idxi_v, acta_v, actb_v, rows_v, zbuf, acc,
              gs0, gs1, as0, as1):
        gsem = (gs0, gs1)
        asem = (as0, as1)
        cid = jax.lax.axis_index("c")
        sid = jax.lax.axis_index("s")
        tb = sid * (ch * ce)

        pltpu.sync_copy(idxj_h.at[sid], idxj_v)
        pltpu.sync_copy(idxi_h.at[sid], idxi_v)

        def zb(r, carry):
            for t in range(GW // 16):
                zbuf[r, pl.ds(16 * t, 16)] = jnp.zeros((16,), jnp.float32)
            return carry
        jax.lax.fori_loop(0, 128, zb, 0)

        def zero_own_rows():
            for q in range(nzc):
                pltpu.sync_copy(zbuf, acc.at[pl.ds(sid * rpt + q * 128, 128)])

        def do_pass(tab_h, col0, act_v, nw, wmap, out_h, c0, c1):
            aw = 16 * nw

            def sf(ci, b):
                pltpu.async_copy(tab_h.at[idxj_v.at[ci]], rows_v.at[b],
                                 gsem[b])
                pltpu.async_copy(
                    act_h.at[pl.ds(tb + ci * ce, ce), pl.ds(col0, aw)],
                    act_v.at[b], asem[b])

            def wf(ci, b):
                pltpu.make_async_copy(tab_h.at[idxj_v.at[ci]], rows_v.at[b],
                                      gsem[b]).wait()
                pltpu.make_async_copy(
                    act_h.at[pl.ds(tb + ci * ce, ce), pl.ds(col0, aw)],
                    act_v.at[b], asem[b]).wait()

            def step(ci, b):
                wf(ci, b)

                def edge(e, c2):
                    wv = [act_v[b, e, pl.ds(16 * t, 16)] for t in range(nw)]
                    for t in range(GW // 16):
                        sl = pl.ds(16 * t, 16)
                        rows_v[b, e, sl] = rows_v[b, e, sl] * wv[wmap[t]]
                    return c2
                jax.lax.fori_loop(0, ce, edge, 0)
                pltpu.sync_copy(rows_v.at[b], acc.at[idxi_v.at[ci]], add=True)

                @pl.when(ci + 2 < c1)
                def _():
                    sf(ci + 2, b)

            length = c1 - c0
            sf(c0, 0)
            if length > 1:
                sf(c0 + 1, 1)

            def pair(ci2, carry):
                step(c0 + 2 * ci2, 0)
                step(c0 + 2 * ci2 + 1, 1)
                return carry
            jax.lax.fori_loop(0, length // 2, pair, 0)
            if length % 2:
                step(c1 - 1, 0)

            plsc.subcore_barrier()
            for q in range(nzc):
                rr = pl.ds(sid * rpt + q * 128, 128)
                pltpu.sync_copy(acc.at[rr], out_h.at[rr])

        zero_own_rows()
        plsc.subcore_barrier()

        @pl.when(cid == 0)
        def _():
            do_pass(tg0_h, 0, acta_v, 4, _WMAP_A, mg0_h, 0, ch)

        @pl.when(cid == 1)
        def _():
            do_pass(tg1_h, NC, acta_v, 4, _WMAP_A, mg1_h, 0, ch)

        zero_own_rows()
        plsc.subcore_barrier()

        @pl.when(cid == 0)
        def _():
            do_pass(tg2_h, 2 * NC, actb_v, 2, _WMAP_B, mg2a_h, 0, ch // 2)

        @pl.when(cid == 1)
        def _():
            do_pass(tg2_h, 2 * NC, actb_v, 2, _WMAP_B, mg2b_h, ch // 2, ch)

    return sc_fn(tg0, tg1, tg2, act, idxi3, idxj3)


_PERM = np.array([3 * (p % NC) + p // NC for p in range(3 * NC)])
_P9 = np.zeros((NCOMP * NC, NCOMP * NC), np.float32)
for _c in range(NC):
    for _k in range(NCOMP):
        _P9[_c * NCOMP + _k, _k * NC + _c] = 1.0


def kernel(X, cij, rbf, idx_i, idx_j, w0, b0, w1, b1, w2, b2,
           wt0, wt1, wt2, wt3, wt4, wt5):
    n = X.shape[0]
    e = rbf.shape[0]

    xc = X.reshape(n, NCOMP * NC)
    ch = e // (NT * 80)
    idxi3 = idx_i.reshape(NT, ch, 80)
    idxj3 = idx_j.reshape(NT, ch, 80)

    p9 = jnp.asarray(_P9)
    act = _mlp(rbf, cij[:, None], w0.T, b0[None, :], w1.T, b1[None, :],
               w2[_PERM].T, b2[_PERM][None, :])
    tg0, tg1, tg2, xn9 = _prep(xc, p9, wt0.T, wt1.T, wt2.T)
    mg0, mg1, mg2a, mg2b = _sc_messages(tg0, tg1, tg2, act, idxi3, idxj3, n)
    oc = _tail(mg0, mg1, mg2a, mg2b, tg0, tg1, tg2, xn9, p9.T,
               wt3.T, wt4.T, wt5.T, n)
    return oc.reshape(n, NC, 3, 3)


# speedup vs baseline: 90.1701x; 1.1213x over previous
import functools

import jax
import jax.numpy as jnp
import numpy as np
from jax.experimental import pallas as pl
from jax.experimental.pallas import tpu as pltpu
from jax.experimental.pallas import tpu_sc as plsc

NC = 32
NCOMP = 9
GW = 3 * NC
BE = 3200
BN = 1000


def _silu(x):
    return x * jax.nn.sigmoid(x)


def _mlp_body(rbf, cij, w0t, b0, w1t, b1, w2t, b2, act):
    h = _silu(jnp.dot(rbf[...], w0t[...]) + b0[...])
    h = _silu(jnp.dot(h, w1t[...]) + b1[...])
    cv = cij[pl.ds(pl.program_id(0) * BE, BE)]
    act[...] = _silu(jnp.dot(h, w2t[...]) + b2[...]) * cv[:, None]


def _mlp(rbf, cij, w0t, b0, w1t, b1, w2t, b2):
    ep = rbf.shape[0]
    full = lambda i: (0, 0)
    blk = lambda i: (i, 0)
    return pl.pallas_call(
        _mlp_body,
        grid=(ep // BE,),
        in_specs=[
            pl.BlockSpec((BE, 8), blk),
            pl.BlockSpec((160000,), lambda i: (0,)),
            pl.BlockSpec((8, NC), full),
            pl.BlockSpec((1, NC), full),
            pl.BlockSpec((NC, 2 * NC), full),
            pl.BlockSpec((1, 2 * NC), full),
            pl.BlockSpec((2 * NC, 3 * NC), full),
            pl.BlockSpec((1, 3 * NC), full),
        ],
        out_specs=[pl.BlockSpec((BE, 3 * NC), blk)],
        out_shape=[jax.ShapeDtypeStruct((ep, 3 * NC), jnp.float32)],
    )(rbf, cij, w0t, b0, w1t, b1, w2t, b2)[0]


def _prep_body(xc, p, wt0t, wt1t, wt2t, tg0, tg1, tg2, xn9):
    x9r = jnp.dot(xc[...], p[...])
    nrm1 = sum(x9r[:, k * NC : (k + 1) * NC] ** 2 for k in range(NCOMP)) + 1.0
    inv = 1.0 / nrm1
    x = [x9r[:, k * NC : (k + 1) * NC] * inv for k in range(NCOMP)]
    xn9[...] = jnp.concatenate(x, 1)
    i_ = (x[0] + x[4] + x[8]) * (1.0 / 3.0)
    a01 = 0.5 * (x[1] - x[3]); a02 = 0.5 * (x[2] - x[6]); a12 = 0.5 * (x[5] - x[7])
    s00 = x[0] - i_; s01 = 0.5 * (x[1] + x[3]); s02 = 0.5 * (x[2] + x[6])
    s11 = x[4] - i_; s12 = 0.5 * (x[5] + x[7])
    w1t_ = wt1t[...]
    w2t_ = wt2t[...]
    tg0[...] = jnp.concatenate(
        [jnp.dot(i_, wt0t[...]), jnp.dot(a01, w1t_), jnp.dot(a02, w1t_)], 1)
    tg1[...] = jnp.concatenate(
        [jnp.dot(a12, w1t_), jnp.dot(s00, w2t_), jnp.dot(s01, w2t_)], 1)
    tg2[...] = jnp.concatenate(
        [jnp.dot(s02, w2t_), jnp.dot(s11, w2t_), jnp.dot(s12, w2t_)], 1)


def _prep(xc, p, wt0t, wt1t, wt2t):
    n = xc.shape[0]
    full = lambda i: (0, 0)
    blk = lambda i: (i, 0)
    return pl.pallas_call(
        _prep_body,
        grid=(n // BN,),
        in_specs=[
            pl.BlockSpec((BN, NCOMP * NC), blk),
            pl.BlockSpec((NCOMP * NC, NCOMP * NC), full),
            pl.BlockSpec((NC, NC), full),
            pl.BlockSpec((NC, NC), full),
            pl.BlockSpec((NC, NC), full),
        ],
        out_specs=[
            pl.BlockSpec((BN, GW), blk),
            pl.BlockSpec((BN, GW), blk),
            pl.BlockSpec((BN, GW), blk),
            pl.BlockSpec((BN, NCOMP * NC), blk),
        ],
        out_shape=[
            jax.ShapeDtypeStruct((n, GW), jnp.float32),
            jax.ShapeDtypeStruct((n, GW), jnp.float32),
            jax.ShapeDtypeStruct((n, GW), jnp.float32),
            jax.ShapeDtypeStruct((n, NCOMP * NC), jnp.float32),
        ],
    )(xc, p, wt0t, wt1t, wt2t)


def _comps9(g0, g1, g2):
    i_, a01, a02 = (g0[:, k * NC : (k + 1) * NC] for k in range(3))
    a12, s00, s01 = (g1[:, k * NC : (k + 1) * NC] for k in range(3))
    s02, s11, s12 = (g2[:, k * NC : (k + 1) * NC] for k in range(3))
    s22 = -(s00 + s11)
    return [[i_ + s00, a01 + s01, a02 + s02],
            [-a01 + s01, i_ + s11, a12 + s12],
            [-a02 + s02, -a12 + s12, i_ + s22]]


def _tail_body(mg0, mg1, mg2a, mg2b, tg0, tg1, tg2, xn9, pt, wt3t, wt4t, wt5t,
               oc):
    m = _comps9(mg0[...], mg1[...], mg2a[...] + mg2b[...])
    y = _comps9(tg0[...], tg1[...], tg2[...])
    C = [[sum(m[r][k] * y[k][c] + y[r][k] * m[k][c] for k in range(3))
          for c in range(3)] for r in range(3)]
    inv = 1.0 / (sum(C[r][c] ** 2 for r in range(3) for c in range(3)) + 1.0)
    ci = (C[0][0] + C[1][1] + C[2][2]) * (1.0 / 3.0)
    ca01 = 0.5 * (C[0][1] - C[1][0]); ca02 = 0.5 * (C[0][2] - C[2][0])
    ca12 = 0.5 * (C[1][2] - C[2][1])
    cs00 = C[0][0] - ci; cs01 = 0.5 * (C[0][1] + C[1][0])
    cs02 = 0.5 * (C[0][2] + C[2][0]); cs11 = C[1][1] - ci
    cs12 = 0.5 * (C[1][2] + C[2][1])
    w3, w4, w5 = wt3t[...], wt4t[...], wt5t[...]
    di = jnp.dot(ci * inv, w3)
    da01 = jnp.dot(ca01 * inv, w4); da02 = jnp.dot(ca02 * inv, w4)
    da12 = jnp.dot(ca12 * inv, w4)
    ds00 = jnp.dot(cs00 * inv, w5); ds01 = jnp.dot(cs01 * inv, w5)
    ds02 = jnp.dot(cs02 * inv, w5); ds11 = jnp.dot(cs11 * inv, w5)
    ds12 = jnp.dot(cs12 * inv, w5)
    ds22 = -(ds00 + ds11)
    d = [[di + ds00, da01 + ds01, da02 + ds02],
         [-da01 + ds01, di + ds11, da12 + ds12],
         [-da02 + ds02, -da12 + ds12, di + ds22]]
    xr = xn9[...]
    out = []
    for r in range(3):
        for c in range(3):
            dd = sum(d[r][k] * d[k][c] for k in range(3))
            out.append(xr[:, (3 * r + c) * NC : (3 * r + c + 1) * NC]
                       + d[r][c] + dd)
    oc[...] = jnp.dot(jnp.concatenate(out, 1), pt[...])


def _tail(mg0, mg1, mg2a, mg2b, tg0, tg1, tg2, xn9, pt, wt3t, wt4t, wt5t, n):
    full = lambda i: (0, 0)
    blk = lambda i: (i, 0)
    return pl.pallas_call(
        _tail_body,
        grid=(n // BN,),
        in_specs=[pl.BlockSpec((BN, GW), blk)] * 7
        + [pl.BlockSpec((BN, NCOMP * NC), blk)]
        + [pl.BlockSpec((NCOMP * NC, NCOMP * NC), full)]
        + [pl.BlockSpec((NC, NC), full)] * 3,
        out_specs=[pl.BlockSpec((BN, NCOMP * NC), blk)],
        out_shape=[jax.ShapeDtypeStruct((n, NCOMP * NC), jnp.float32)],
    )(mg0, mg1, mg2a, mg2b, tg0, tg1, tg2, xn9, pt, wt3t, wt4t, wt5t)[0]


NT = 16
_WMAP_A = [0, 1, 2, 3, 2, 3]
_WMAP_B = [0, 1, 0, 1, 0, 1]


def _sc_messages(tg0, tg1, tg2, act, idxi2, idxj2, n):
    ch = idxi2.shape[0] // NT
    ce = idxi2.shape[1]
    na = ((n + NT * 128 - 1) // (NT * 128)) * (NT * 128)
    rpt = na // NT
    nzc = rpt // 128

    mesh = plsc.VectorSubcoreMesh(core_axis_name="c", subcore_axis_name="s")
    mshape = jax.ShapeDtypeStruct((na, GW), jnp.float32)

    @functools.partial(
        pl.kernel, mesh=mesh,
        compiler_params=pltpu.CompilerParams(use_tc_tiling_on_sc=False),
        out_type=[mshape, mshape, mshape, mshape],
        scratch_types=[
            pltpu.VMEM((ch, ce), jnp.int32),
            pltpu.VMEM((ch, ce), jnp.int32),
            pltpu.VMEM((2, ce, 4 * 16), jnp.float32),
            pltpu.VMEM((2, ce, 2 * 16), jnp.float32),
            pltpu.VMEM((2, ce, GW), jnp.float32),
            pltpu.VMEM((128, GW), jnp.float32),
            pltpu.VMEM_SHARED((na, GW), jnp.float32),
            pltpu.SemaphoreType.DMA,
            pltpu.SemaphoreType.DMA,
            pltpu.SemaphoreType.DMA,
            pltpu.SemaphoreType.DMA,
        ],
    )
    def sc_fn(tg0_h, tg1_h, tg2_h, act_h, idxi_h, idxj_h,
              mg0_h, mg1_h, mg2a_h, mg2b_h,
              idxj_v, idxi_v, acta_v, actb_v, rows_v, zbuf, acc,
              gs0, gs1, as0, as1):
        gsem = (gs0, gs1)
        asem = (as0, as1)
        cid = jax.lax.axis_index("c")
        sid = jax.lax.axis_index("s")
        tb = sid * (ch * ce)

        pltpu.sync_copy(idxj_h.at[pl.ds(sid * ch, ch)], idxj_v)
        pltpu.sync_copy(idxi_h.at[pl.ds(sid * ch, ch)], idxi_v)

        def zb(r, carry):
            for t in range(GW // 16):
                zbuf[r, pl.ds(16 * t, 16)] = jnp.zeros((16,), jnp.float32)
            return carry
        jax.lax.fori_loop(0, 128, zb, 0)

        def zero_own_rows():
            for q in range(nzc):
                pltpu.sync_copy(zbuf, acc.at[pl.ds(sid * rpt + q * 128, 128)])

        def do_pass(tab_h, col0, act_v, nw, wmap, out_h, c0, c1):
            aw = 16 * nw

            def sf(ci, b):
                pltpu.async_copy(tab_h.at[idxj_v.at[ci]], rows_v.at[b],
                                 gsem[b])
                pltpu.async_copy(
                    act_h.at[pl.ds(tb + ci * ce, ce), pl.ds(col0, aw)],
                    act_v.at[b], asem[b])

            def wf(ci, b):
                pltpu.make_async_copy(tab_h.at[idxj_v.at[ci]], rows_v.at[b],
                                      gsem[b]).wait()
                pltpu.make_async_copy(
                    act_h.at[pl.ds(tb + ci * ce, ce), pl.ds(col0, aw)],
                    act_v.at[b], asem[b]).wait()

            def step(ci, b):
                wf(ci, b)

                def edge(e, c2):
                    wv = [act_v[b, e, pl.ds(16 * t, 16)]
                          for t in range(nw)]
                    for t in range(GW // 16):
                        sl = pl.ds(16 * t, 16)
                        rows_v[b, e, sl] = rows_v[b, e, sl] * wv[wmap[t]]
                    return c2
                jax.lax.fori_loop(0, ce, edge, 0)
                pltpu.sync_copy(rows_v.at[b], acc.at[idxi_v.at[ci]], add=True)

                @pl.when(ci + 2 < c1)
                def _():
                    sf(ci + 2, b)

            length = c1 - c0
            sf(c0, 0)
            if length > 1:
                sf(c0 + 1, 1)

            def pair(ci2, carry):
                step(c0 + 2 * ci2, 0)
                step(c0 + 2 * ci2 + 1, 1)
                return carry
            jax.lax.fori_loop(0, length // 2, pair, 0)
            if length % 2:
                step(c1 - 1, 0)

            plsc.subcore_barrier()
            for q in range(nzc):
                rr = pl.ds(sid * rpt + q * 128, 128)
                pltpu.sync_copy(acc.at[rr], out_h.at[rr])

        zero_own_rows()
        plsc.subcore_barrier()

        @pl.when(cid == 0)
        def _():
            do_pass(tg0_h, 0, acta_v, 4, _WMAP_A, mg0_h, 0, ch)

        @pl.when(cid == 1)
        def _():
            do_pass(tg1_h, NC, acta_v, 4, _WMAP_A, mg1_h, 0, ch)

        zero_own_rows()
        plsc.subcore_barrier()

        @pl.when(cid == 0)
        def _():
            do_pass(tg2_h, 2 * NC, actb_v, 2, _WMAP_B, mg2a_h, 0, ch // 2)

        @pl.when(cid == 1)
        def _():
            do_pass(tg2_h, 2 * NC, actb_v, 2, _WMAP_B, mg2b_h, ch // 2, ch)

    return sc_fn(tg0, tg1, tg2, act, idxi2, idxj2)


_PERM = np.array([3 * (p % NC) + p // NC for p in range(3 * NC)])
_P9 = np.zeros((NCOMP * NC, NCOMP * NC), np.float32)
for _c in range(NC):
    for _k in range(NCOMP):
        _P9[_c * NCOMP + _k, _k * NC + _c] = 1.0


def kernel(X, cij, rbf, idx_i, idx_j, w0, b0, w1, b1, w2, b2,
           wt0, wt1, wt2, wt3, wt4, wt5):
    n = X.shape[0]
    e = rbf.shape[0]

    xc = X.reshape(n, NCOMP * NC)
    ch = e // (NT * 80)
    idxi2 = idx_i.reshape(NT * ch, 80)
    idxj2 = idx_j.reshape(NT * ch, 80)

    p9 = jnp.asarray(_P9)
    act = _mlp(rbf, cij, w0.T, b0[None, :], w1.T, b1[None, :],
               w2[_PERM].T, b2[_PERM][None, :])
    tg0, tg1, tg2, xn9 = _prep(xc, p9, wt0.T, wt1.T, wt2.T)
    mg0, mg1, mg2a, mg2b = _sc_messages(tg0, tg1, tg2, act, idxi2, idxj2, n)
    oc = _tail(mg0, mg1, mg2a, mg2b, tg0, tg1, tg2, xn9, p9.T,
               wt3.T, wt4.T, wt5.T, n)
    return oc.reshape(n, NC, 3, 3)


# speedup vs baseline: 102.2893x; 1.1344x over previous
import functools

import jax
import jax.numpy as jnp
import numpy as np
from jax.experimental import pallas as pl
from jax.experimental.pallas import tpu as pltpu
from jax.experimental.pallas import tpu_sc as plsc

NC = 32
NCOMP = 9
GW = 3 * NC
BE = 3200
BN = 1000


def _silu(x):
    return x * jax.nn.sigmoid(x)


def _mlp_body(rbf, cij, w0t, b0, w1t, b1, w2t, b2, act):
    h = _silu(jnp.dot(rbf[...], w0t[...]) + b0[...])
    h = _silu(jnp.dot(h, w1t[...]) + b1[...])
    cv = cij[pl.ds(pl.program_id(0) * BE, BE)]
    a = _silu(jnp.dot(h, w2t[...]) + b2[...]) * cv[:, None]
    act[...] = jnp.concatenate([a, jnp.zeros((BE, 32), jnp.float32)], 1)


def _mlp(rbf, cij, w0t, b0, w1t, b1, w2t, b2):
    ep = rbf.shape[0]
    full = lambda i: (0, 0)
    blk = lambda i: (i, 0)
    return pl.pallas_call(
        _mlp_body,
        grid=(ep // BE,),
        in_specs=[
            pl.BlockSpec((BE, 8), blk),
            pl.BlockSpec((160000,), lambda i: (0,)),
            pl.BlockSpec((8, NC), full),
            pl.BlockSpec((1, NC), full),
            pl.BlockSpec((NC, 2 * NC), full),
            pl.BlockSpec((1, 2 * NC), full),
            pl.BlockSpec((2 * NC, 3 * NC), full),
            pl.BlockSpec((1, 3 * NC), full),
        ],
        out_specs=[pl.BlockSpec((BE, 128), blk)],
        out_shape=[jax.ShapeDtypeStruct((ep, 128), jnp.float32)],
    )(rbf, cij, w0t, b0, w1t, b1, w2t, b2)[0]


def _prep_body(xc, p, wt0t, wt1t, wt2t, tg0, tg1, tg2, xn9):
    x9r = jnp.dot(xc[...], p[...])
    nrm1 = sum(x9r[:, k * NC : (k + 1) * NC] ** 2 for k in range(NCOMP)) + 1.0
    inv = 1.0 / nrm1
    x = [x9r[:, k * NC : (k + 1) * NC] * inv for k in range(NCOMP)]
    xn9[...] = jnp.concatenate(x, 1)
    i_ = (x[0] + x[4] + x[8]) * (1.0 / 3.0)
    a01 = 0.5 * (x[1] - x[3]); a02 = 0.5 * (x[2] - x[6]); a12 = 0.5 * (x[5] - x[7])
    s00 = x[0] - i_; s01 = 0.5 * (x[1] + x[3]); s02 = 0.5 * (x[2] + x[6])
    s11 = x[4] - i_; s12 = 0.5 * (x[5] + x[7])
    w1t_ = wt1t[...]
    w2t_ = wt2t[...]
    tg0[...] = jnp.concatenate(
        [jnp.dot(i_, wt0t[...]), jnp.dot(a01, w1t_), jnp.dot(a02, w1t_)], 1)
    tg1[...] = jnp.concatenate(
        [jnp.dot(a12, w1t_), jnp.dot(s00, w2t_), jnp.dot(s01, w2t_)], 1)
    tg2[...] = jnp.concatenate(
        [jnp.dot(s02, w2t_), jnp.dot(s11, w2t_), jnp.dot(s12, w2t_)], 1)


def _prep(xc, p, wt0t, wt1t, wt2t):
    n = xc.shape[0]
    full = lambda i: (0, 0)
    blk = lambda i: (i, 0)
    return pl.pallas_call(
        _prep_body,
        grid=(n // BN,),
        in_specs=[
            pl.BlockSpec((BN, NCOMP * NC), blk),
            pl.BlockSpec((NCOMP * NC, NCOMP * NC), full),
            pl.BlockSpec((NC, NC), full),
            pl.BlockSpec((NC, NC), full),
            pl.BlockSpec((NC, NC), full),
        ],
        out_specs=[
            pl.BlockSpec((BN, GW), blk),
            pl.BlockSpec((BN, GW), blk),
            pl.BlockSpec((BN, GW), blk),
            pl.BlockSpec((BN, NCOMP * NC), blk),
        ],
        out_shape=[
            jax.ShapeDtypeStruct((n, GW), jnp.float32),
            jax.ShapeDtypeStruct((n, GW), jnp.float32),
            jax.ShapeDtypeStruct((n, GW), jnp.float32),
            jax.ShapeDtypeStruct((n, NCOMP * NC), jnp.float32),
        ],
    )(xc, p, wt0t, wt1t, wt2t)


def _comps9(g0, g1, g2):
    i_, a01, a02 = (g0[:, k * NC : (k + 1) * NC] for k in range(3))
    a12, s00, s01 = (g1[:, k * NC : (k + 1) * NC] for k in range(3))
    s02, s11, s12 = (g2[:, k * NC : (k + 1) * NC] for k in range(3))
    s22 = -(s00 + s11)
    return [[i_ + s00, a01 + s01, a02 + s02],
            [-a01 + s01, i_ + s11, a12 + s12],
            [-a02 + s02, -a12 + s12, i_ + s22]]


def _tail_body(mg0, mg1, mg2a, mg2b, tg0, tg1, tg2, xn9, pt, wt3t, wt4t, wt5t,
               oc):
    m = _comps9(mg0[...], mg1[...], mg2a[...] + mg2b[...])
    y = _comps9(tg0[...], tg1[...], tg2[...])
    C = [[sum(m[r][k] * y[k][c] + y[r][k] * m[k][c] for k in range(3))
          for c in range(3)] for r in range(3)]
    inv = 1.0 / (sum(C[r][c] ** 2 for r in range(3) for c in range(3)) + 1.0)
    ci = (C[0][0] + C[1][1] + C[2][2]) * (1.0 / 3.0)
    ca01 = 0.5 * (C[0][1] - C[1][0]); ca02 = 0.5 * (C[0][2] - C[2][0])
    ca12 = 0.5 * (C[1][2] - C[2][1])
    cs00 = C[0][0] - ci; cs01 = 0.5 * (C[0][1] + C[1][0])
    cs02 = 0.5 * (C[0][2] + C[2][0]); cs11 = C[1][1] - ci
    cs12 = 0.5 * (C[1][2] + C[2][1])
    w3, w4, w5 = wt3t[...], wt4t[...], wt5t[...]
    di = jnp.dot(ci * inv, w3)
    da01 = jnp.dot(ca01 * inv, w4); da02 = jnp.dot(ca02 * inv, w4)
    da12 = jnp.dot(ca12 * inv, w4)
    ds00 = jnp.dot(cs00 * inv, w5); ds01 = jnp.dot(cs01 * inv, w5)
    ds02 = jnp.dot(cs02 * inv, w5); ds11 = jnp.dot(cs11 * inv, w5)
    ds12 = jnp.dot(cs12 * inv, w5)
    ds22 = -(ds00 + ds11)
    d = [[di + ds00, da01 + ds01, da02 + ds02],
         [-da01 + ds01, di + ds11, da12 + ds12],
         [-da02 + ds02, -da12 + ds12, di + ds22]]
    xr = xn9[...]
    out = []
    for r in range(3):
        for c in range(3):
            dd = sum(d[r][k] * d[k][c] for k in range(3))
            out.append(xr[:, (3 * r + c) * NC : (3 * r + c + 1) * NC]
                       + d[r][c] + dd)
    oc[...] = jnp.dot(jnp.concatenate(out, 1), pt[...])


def _tail(mg0, mg1, mg2a, mg2b, tg0, tg1, tg2, xn9, pt, wt3t, wt4t, wt5t, n):
    full = lambda i: (0, 0)
    blk = lambda i: (i, 0)
    return pl.pallas_call(
        _tail_body,
        grid=(n // BN,),
        in_specs=[pl.BlockSpec((BN, GW), blk)] * 7
        + [pl.BlockSpec((BN, NCOMP * NC), blk)]
        + [pl.BlockSpec((NCOMP * NC, NCOMP * NC), full)]
        + [pl.BlockSpec((NC, NC), full)] * 3,
        out_specs=[pl.BlockSpec((BN, NCOMP * NC), blk)],
        out_shape=[jax.ShapeDtypeStruct((n, NCOMP * NC), jnp.float32)],
    )(mg0, mg1, mg2a, mg2b, tg0, tg1, tg2, xn9, pt, wt3t, wt4t, wt5t)[0]


NT = 16
_WMAP_A = [0, 1, 2, 3, 2, 3]
_WMAP_B = [0, 1, 0, 1, 0, 1]


def _sc_messages(tg0, tg1, tg2, act, idxi2, idxj2, n):
    ch = idxi2.shape[0] // NT
    ce = idxi2.shape[1]
    na = ((n + NT * 128 - 1) // (NT * 128)) * (NT * 128)
    rpt = na // NT
    nzc = rpt // 128

    mesh = plsc.VectorSubcoreMesh(core_axis_name="c", subcore_axis_name="s")
    mshape = jax.ShapeDtypeStruct((na, GW), jnp.float32)

    @functools.partial(
        pl.kernel, mesh=mesh,
        compiler_params=pltpu.CompilerParams(use_tc_tiling_on_sc=False),
        out_type=[mshape, mshape, mshape, mshape],
        scratch_types=[
            pltpu.VMEM((ch, ce), jnp.int32),
            pltpu.VMEM((ch, ce), jnp.int32),
            pltpu.VMEM((2, ce, 4 * 16), jnp.float32),
            pltpu.VMEM((2, ce, 2 * 16), jnp.float32),
            pltpu.VMEM((2, ce, GW), jnp.float32),
            pltpu.VMEM((128, GW), jnp.float32),
            pltpu.VMEM_SHARED((na, GW), jnp.float32),
            pltpu.SemaphoreType.DMA,
            pltpu.SemaphoreType.DMA,
            pltpu.SemaphoreType.DMA,
            pltpu.SemaphoreType.DMA,
        ],
    )
    def sc_fn(tg0_h, tg1_h, tg2_h, act_h, idxi_h, idxj_h,
              mg0_h, mg1_h, mg2a_h, mg2b_h,
              idxj_v, idxi_v, acta_v, actb_v, rows_v, zbuf, acc,
              gs0, gs1, as0, as1):
        gsem = (gs0, gs1)
        asem = (as0, as1)
        cid = jax.lax.axis_index("c")
        sid = jax.lax.axis_index("s")
        tb = sid * (ch * ce)

        pltpu.sync_copy(idxj_h.at[pl.ds(sid * ch, ch)], idxj_v)
        pltpu.sync_copy(idxi_h.at[pl.ds(sid * ch, ch)], idxi_v)

        def zb(r, carry):
            for t in range(GW // 16):
                zbuf[r, pl.ds(16 * t, 16)] = jnp.zeros((16,), jnp.float32)
            return carry
        jax.lax.fori_loop(0, 128, zb, 0)

        def zero_own_rows():
            for q in range(nzc):
                pltpu.sync_copy(zbuf, acc.at[pl.ds(sid * rpt + q * 128, 128)])

        def do_pass(tab_h, col0, act_v, nw, wmap, out_h, c0, c1):
            aw = 16 * nw

            def sf(ci, b):
                pltpu.async_copy(tab_h.at[idxj_v.at[ci]], rows_v.at[b],
                                 gsem[b])
                pltpu.async_copy(
                    act_h.at[pl.ds(tb + ci * ce, ce), pl.ds(col0, aw)],
                    act_v.at[b], asem[b])

            def wf(ci, b):
                pltpu.make_async_copy(tab_h.at[idxj_v.at[ci]], rows_v.at[b],
                                      gsem[b]).wait()
                pltpu.make_async_copy(
                    act_h.at[pl.ds(tb + ci * ce, ce), pl.ds(col0, aw)],
                    act_v.at[b], asem[b]).wait()

            def step(ci, b):
                wf(ci, b)

                def edge(e, c2):
                    wv = [act_v[b, e, pl.ds(16 * t, 16)]
                          for t in range(nw)]
                    for t in range(GW // 16):
                        sl = pl.ds(16 * t, 16)
                        rows_v[b, e, sl] = rows_v[b, e, sl] * wv[wmap[t]]
                    return c2
                jax.lax.fori_loop(0, ce, edge, 0)
                pltpu.sync_copy(rows_v.at[b], acc.at[idxi_v.at[ci]], add=True)

                @pl.when(ci + 2 < c1)
                def _():
                    sf(ci + 2, b)

            length = c1 - c0
            sf(c0, 0)
            if length > 1:
                sf(c0 + 1, 1)

            def pair(ci2, carry):
                step(c0 + 2 * ci2, 0)
                step(c0 + 2 * ci2 + 1, 1)
                return carry
            jax.lax.fori_loop(0, length // 2, pair, 0)
            if length % 2:
                step(c1 - 1, 0)

            plsc.subcore_barrier()
            for q in range(nzc):
                rr = pl.ds(sid * rpt + q * 128, 128)
                pltpu.sync_copy(acc.at[rr], out_h.at[rr])

        zero_own_rows()
        plsc.subcore_barrier()

        @pl.when(cid == 0)
        def _():
            do_pass(tg0_h, 0, acta_v, 4, _WMAP_A, mg0_h, 0, ch)

        @pl.when(cid == 1)
        def _():
            do_pass(tg1_h, NC, acta_v, 4, _WMAP_A, mg1_h, 0, ch)

        zero_own_rows()
        plsc.subcore_barrier()

        @pl.when(cid == 0)
        def _():
            do_pass(tg2_h, 2 * NC, actb_v, 2, _WMAP_B, mg2a_h, 0, ch // 2)

        @pl.when(cid == 1)
        def _():
            do_pass(tg2_h, 2 * NC, actb_v, 2, _WMAP_B, mg2b_h, ch // 2, ch)

    return sc_fn(tg0, tg1, tg2, act, idxi2, idxj2)


_PERM = np.array([3 * (p % NC) + p // NC for p in range(3 * NC)])
_P9 = np.zeros((NCOMP * NC, NCOMP * NC), np.float32)
for _c in range(NC):
    for _k in range(NCOMP):
        _P9[_c * NCOMP + _k, _k * NC + _c] = 1.0


def kernel(X, cij, rbf, idx_i, idx_j, w0, b0, w1, b1, w2, b2,
           wt0, wt1, wt2, wt3, wt4, wt5):
    n = X.shape[0]
    e = rbf.shape[0]

    xc = X.reshape(n, NCOMP * NC)
    ch = e // (NT * 80)
    idxi2 = idx_i.reshape(NT * ch, 80)
    idxj2 = idx_j.reshape(NT * ch, 80)

    p9 = jnp.asarray(_P9)
    act = _mlp(rbf, cij, w0.T, b0[None, :], w1.T, b1[None, :],
               w2[_PERM].T, b2[_PERM][None, :])
    tg0, tg1, tg2, xn9 = _prep(xc, p9, wt0.T, wt1.T, wt2.T)
    mg0, mg1, mg2a, mg2b = _sc_messages(tg0, tg1, tg2, act, idxi2, idxj2, n)
    oc = _tail(mg0, mg1, mg2a, mg2b, tg0, tg1, tg2, xn9, p9.T,
               wt3.T, wt4.T, wt5.T, n)
    return oc.reshape(n, NC, 3, 3)


# speedup vs baseline: 105.5074x; 1.0315x over previous
import functools

import jax
import jax.numpy as jnp
import numpy as np
from jax.experimental import pallas as pl
from jax.experimental.pallas import tpu as pltpu
from jax.experimental.pallas import tpu_sc as plsc

NC = 32
NCOMP = 9
GW = 3 * NC
BE = 3200
BN = 1000


def _silu(x):
    return x * jax.nn.sigmoid(x)


def _mlp_body(rbf, cij, w0t, b0, w1t, b1, w2t, b2, act):
    h = _silu(jnp.dot(rbf[...], w0t[...]) + b0[...])
    h = _silu(jnp.dot(h, w1t[...]) + b1[...])
    cv = cij[pl.ds(pl.program_id(0) * BE, BE)]
    a = _silu(jnp.dot(h, w2t[...]) + b2[...]) * cv[:, None]
    act[...] = jnp.concatenate([a, jnp.zeros((BE, 32), jnp.float32)], 1)


def _mlp(rbf, cij, w0t, b0, w1t, b1, w2t, b2):
    ep = rbf.shape[0]
    full = lambda i: (0, 0)
    blk = lambda i: (i, 0)
    return pl.pallas_call(
        _mlp_body,
        grid=(ep // BE,),
        in_specs=[
            pl.BlockSpec((BE, 8), blk),
            pl.BlockSpec((160000,), lambda i: (0,)),
            pl.BlockSpec((8, NC), full),
            pl.BlockSpec((1, NC), full),
            pl.BlockSpec((NC, 2 * NC), full),
            pl.BlockSpec((1, 2 * NC), full),
            pl.BlockSpec((2 * NC, 3 * NC), full),
            pl.BlockSpec((1, 3 * NC), full),
        ],
        out_specs=[pl.BlockSpec((BE, 128), blk)],
        out_shape=[jax.ShapeDtypeStruct((ep, 128), jnp.float32)],
    )(rbf, cij, w0t, b0, w1t, b1, w2t, b2)[0]


def _prep_body(xc, p, wt0t, wt1t, wt2t, tg0, tg1, tg2, xn9):
    x9r = jnp.dot(xc[...], p[...])
    nrm1 = sum(x9r[:, k * NC : (k + 1) * NC] ** 2 for k in range(NCOMP)) + 1.0
    inv = 1.0 / nrm1
    x = [x9r[:, k * NC : (k + 1) * NC] * inv for k in range(NCOMP)]
    xn9[...] = jnp.concatenate(x, 1)
    i_ = (x[0] + x[4] + x[8]) * (1.0 / 3.0)
    a01 = 0.5 * (x[1] - x[3]); a02 = 0.5 * (x[2] - x[6]); a12 = 0.5 * (x[5] - x[7])
    s00 = x[0] - i_; s01 = 0.5 * (x[1] + x[3]); s02 = 0.5 * (x[2] + x[6])
    s11 = x[4] - i_; s12 = 0.5 * (x[5] + x[7])
    w1t_ = wt1t[...]
    w2t_ = wt2t[...]
    tg0[...] = jnp.concatenate(
        [jnp.dot(i_, wt0t[...]), jnp.dot(a01, w1t_), jnp.dot(a02, w1t_)], 1)
    tg1[...] = jnp.concatenate(
        [jnp.dot(a12, w1t_), jnp.dot(s00, w2t_), jnp.dot(s01, w2t_)], 1)
    tg2[...] = jnp.concatenate(
        [jnp.dot(s02, w2t_), jnp.dot(s11, w2t_), jnp.dot(s12, w2t_)], 1)


def _prep(xc, p, wt0t, wt1t, wt2t):
    n = xc.shape[0]
    full = lambda i: (0, 0)
    blk = lambda i: (i, 0)
    return pl.pallas_call(
        _prep_body,
        grid=(n // BN,),
        in_specs=[
            pl.BlockSpec((BN, NCOMP * NC), blk),
            pl.BlockSpec((NCOMP * NC, NCOMP * NC), full),
            pl.BlockSpec((NC, NC), full),
            pl.BlockSpec((NC, NC), full),
            pl.BlockSpec((NC, NC), full),
        ],
        out_specs=[
            pl.BlockSpec((BN, GW), blk),
            pl.BlockSpec((BN, GW), blk),
            pl.BlockSpec((BN, GW), blk),
            pl.BlockSpec((BN, NCOMP * NC), blk),
        ],
        out_shape=[
            jax.ShapeDtypeStruct((n, GW), jnp.float32),
            jax.ShapeDtypeStruct((n, GW), jnp.float32),
            jax.ShapeDtypeStruct((n, GW), jnp.float32),
            jax.ShapeDtypeStruct((n, NCOMP * NC), jnp.float32),
        ],
    )(xc, p, wt0t, wt1t, wt2t)


def _comps9(g0, g1, g2):
    i_, a01, a02 = (g0[:, k * NC : (k + 1) * NC] for k in range(3))
    a12, s00, s01 = (g1[:, k * NC : (k + 1) * NC] for k in range(3))
    s02, s11, s12 = (g2[:, k * NC : (k + 1) * NC] for k in range(3))
    s22 = -(s00 + s11)
    return [[i_ + s00, a01 + s01, a02 + s02],
            [-a01 + s01, i_ + s11, a12 + s12],
            [-a02 + s02, -a12 + s12, i_ + s22]]


def _tail_body(mg0, mg1, mg2a, mg2b, tg0, tg1, tg2, xn9, pt, wt3t, wt4t, wt5t,
               oc):
    m = _comps9(mg0[...][:, :GW], mg1[...][:, :GW],
                mg2a[...][:, :GW] + mg2b[...][:, :GW])
    y = _comps9(tg0[...], tg1[...], tg2[...])
    C = [[sum(m[r][k] * y[k][c] + y[r][k] * m[k][c] for k in range(3))
          for c in range(3)] for r in range(3)]
    inv = 1.0 / (sum(C[r][c] ** 2 for r in range(3) for c in range(3)) + 1.0)
    ci = (C[0][0] + C[1][1] + C[2][2]) * (1.0 / 3.0)
    ca01 = 0.5 * (C[0][1] - C[1][0]); ca02 = 0.5 * (C[0][2] - C[2][0])
    ca12 = 0.5 * (C[1][2] - C[2][1])
    cs00 = C[0][0] - ci; cs01 = 0.5 * (C[0][1] + C[1][0])
    cs02 = 0.5 * (C[0][2] + C[2][0]); cs11 = C[1][1] - ci
    cs12 = 0.5 * (C[1][2] + C[2][1])
    w3, w4, w5 = wt3t[...], wt4t[...], wt5t[...]
    di = jnp.dot(ci * inv, w3)
    da01 = jnp.dot(ca01 * inv, w4); da02 = jnp.dot(ca02 * inv, w4)
    da12 = jnp.dot(ca12 * inv, w4)
    ds00 = jnp.dot(cs00 * inv, w5); ds01 = jnp.dot(cs01 * inv, w5)
    ds02 = jnp.dot(cs02 * inv, w5); ds11 = jnp.dot(cs11 * inv, w5)
    ds12 = jnp.dot(cs12 * inv, w5)
    ds22 = -(ds00 + ds11)
    d = [[di + ds00, da01 + ds01, da02 + ds02],
         [-da01 + ds01, di + ds11, da12 + ds12],
         [-da02 + ds02, -da12 + ds12, di + ds22]]
    xr = xn9[...]
    out = []
    for r in range(3):
        for c in range(3):
            dd = sum(d[r][k] * d[k][c] for k in range(3))
            out.append(xr[:, (3 * r + c) * NC : (3 * r + c + 1) * NC]
                       + d[r][c] + dd)
    oc[...] = jnp.dot(jnp.concatenate(out, 1), pt[...])


def _tail(mg0, mg1, mg2a, mg2b, tg0, tg1, tg2, xn9, pt, wt3t, wt4t, wt5t, n):
    full = lambda i: (0, 0)
    blk = lambda i: (i, 0)
    return pl.pallas_call(
        _tail_body,
        grid=(n // BN,),
        in_specs=[pl.BlockSpec((BN, 128), blk)] * 4
        + [pl.BlockSpec((BN, GW), blk)] * 3
        + [pl.BlockSpec((BN, NCOMP * NC), blk)]
        + [pl.BlockSpec((NCOMP * NC, NCOMP * NC), full)]
        + [pl.BlockSpec((NC, NC), full)] * 3,
        out_specs=[pl.BlockSpec((BN, NCOMP * NC), blk)],
        out_shape=[jax.ShapeDtypeStruct((n, NCOMP * NC), jnp.float32)],
    )(mg0, mg1, mg2a, mg2b, tg0, tg1, tg2, xn9, pt, wt3t, wt4t, wt5t)[0]


NT = 16
_WMAP_A = [0, 1, 2, 3, 2, 3]
_WMAP_B = [0, 1, 0, 1, 0, 1]


def _sc_messages(tg0, tg1, tg2, act, idx_i, idx_j, n, ce=80):
    ch = idx_i.shape[0] // (NT * ce)
    na = ((n + NT * 128 - 1) // (NT * 128)) * (NT * 128)
    rpt = na // NT
    nzc = rpt // 128

    mesh = plsc.VectorSubcoreMesh(core_axis_name="c", subcore_axis_name="s")
    mshape = jax.ShapeDtypeStruct((na, 128), jnp.float32)

    @functools.partial(
        pl.kernel, mesh=mesh,
        compiler_params=pltpu.CompilerParams(use_tc_tiling_on_sc=False),
        out_type=[mshape, mshape, mshape, mshape],
        scratch_types=[
            pltpu.VMEM((ch * ce,), jnp.int32),
            pltpu.VMEM((2, ce), jnp.int32),
            pltpu.VMEM((2, ce, 4 * 16), jnp.float32),
            pltpu.VMEM((2, ce, 2 * 16), jnp.float32),
            pltpu.VMEM((2, ce, GW), jnp.float32),
            pltpu.VMEM((128, GW), jnp.float32),
            pltpu.VMEM_SHARED((na, GW), jnp.float32),
            pltpu.SemaphoreType.DMA,
            pltpu.SemaphoreType.DMA,
            pltpu.SemaphoreType.DMA,
            pltpu.SemaphoreType.DMA,
            pltpu.SemaphoreType.DMA,
            pltpu.SemaphoreType.DMA,
        ],
    )
    def sc_fn(tg0_h, tg1_h, tg2_h, act_h, idxi_h, idxj_h,
              mg0_h, mg1_h, mg2a_h, mg2b_h,
              idxj_v, idxi_v, acta_v, actb_v, rows_v, zbuf, acc,
              gs0, gs1, as0, as1, is0, is1):
        gsem = (gs0, gs1)
        asem = (as0, as1)
        isem = (is0, is1)
        cid = jax.lax.axis_index("c")
        sid = jax.lax.axis_index("s")
        tb = sid * (ch * ce)

        pltpu.sync_copy(idxj_h.at[pl.ds(tb, ch * ce)], idxj_v)

        def zb(r, carry):
            for t in range(GW // 16):
                zbuf[r, pl.ds(16 * t, 16)] = jnp.zeros((16,), jnp.float32)
            return carry
        jax.lax.fori_loop(0, 128, zb, 0)

        def zero_own_rows():
            for q in range(nzc):
                pltpu.sync_copy(zbuf, acc.at[pl.ds(sid * rpt + q * 128, 128)])

        def do_pass(tab_h, col0, act_v, nw, wmap, out_h, c0, c1):
            aw = 16 * nw

            def sf(ci, b):
                pltpu.async_copy(tab_h.at[idxj_v.at[pl.ds(ci * ce, ce)]],
                                 rows_v.at[b], gsem[b])
                pltpu.async_copy(
                    act_h.at[pl.ds(tb + ci * ce, ce), pl.ds(col0, aw)],
                    act_v.at[b], asem[b])
                pltpu.async_copy(idxi_h.at[pl.ds(tb + ci * ce, ce)],
                                 idxi_v.at[b], isem[b])

            def wf(ci, b):
                pltpu.make_async_copy(tab_h.at[idxj_v.at[pl.ds(ci * ce, ce)]],
                                      rows_v.at[b], gsem[b]).wait()
                pltpu.make_async_copy(
                    act_h.at[pl.ds(tb + ci * ce, ce), pl.ds(col0, aw)],
                    act_v.at[b], asem[b]).wait()
                pltpu.make_async_copy(idxi_h.at[pl.ds(tb + ci * ce, ce)],
                                      idxi_v.at[b], isem[b]).wait()

            def step(ci, b):
                wf(ci, b)

                def edge(e, c2):
                    wv = [act_v[b, e, pl.ds(16 * t, 16)]
                          for t in range(nw)]
                    for t in range(GW // 16):
                        sl = pl.ds(16 * t, 16)
                        rows_v[b, e, sl] = rows_v[b, e, sl] * wv[wmap[t]]
                    return c2
                jax.lax.fori_loop(0, ce, edge, 0)
                pltpu.sync_copy(rows_v.at[b], acc.at[idxi_v.at[b]], add=True)

                @pl.when(ci + 2 < c1)
                def _():
                    sf(ci + 2, b)

            length = c1 - c0
            sf(c0, 0)
            if length > 1:
                sf(c0 + 1, 1)

            def pair(ci2, carry):
                step(c0 + 2 * ci2, 0)
                step(c0 + 2 * ci2 + 1, 1)
                return carry
            jax.lax.fori_loop(0, length // 2, pair, 0)
            if length % 2:
                step(c1 - 1, 0)

            plsc.subcore_barrier()
            for q in range(nzc):
                rr = pl.ds(sid * rpt + q * 128, 128)
                pltpu.sync_copy(acc.at[rr], out_h.at[rr, pl.ds(0, GW)])

        zero_own_rows()
        plsc.subcore_barrier()

        @pl.when(cid == 0)
        def _():
            do_pass(tg0_h, 0, acta_v, 4, _WMAP_A, mg0_h, 0, ch)

        @pl.when(cid == 1)
        def _():
            do_pass(tg1_h, NC, acta_v, 4, _WMAP_A, mg1_h, 0, ch)

        zero_own_rows()
        plsc.subcore_barrier()

        @pl.when(cid == 0)
        def _():
            do_pass(tg2_h, 2 * NC, actb_v, 2, _WMAP_B, mg2a_h, 0, ch // 2)

        @pl.when(cid == 1)
        def _():
            do_pass(tg2_h, 2 * NC, actb_v, 2, _WMAP_B, mg2b_h, ch // 2, ch)

    return sc_fn(tg0, tg1, tg2, act, idx_i, idx_j)


_PERM = np.array([3 * (p % NC) + p // NC for p in range(3 * NC)])
_P9 = np.zeros((NCOMP * NC, NCOMP * NC), np.float32)
for _c in range(NC):
    for _k in range(NCOMP):
        _P9[_c * NCOMP + _k, _k * NC + _c] = 1.0


def kernel(X, cij, rbf, idx_i, idx_j, w0, b0, w1, b1, w2, b2,
           wt0, wt1, wt2, wt3, wt4, wt5):
    n = X.shape[0]
    e = rbf.shape[0]

    xc = X.reshape(n, NCOMP * NC)

    p9 = jnp.asarray(_P9)
    act = _mlp(rbf, cij, w0.T, b0[None, :], w1.T, b1[None, :],
               w2[_PERM].T, b2[_PERM][None, :])
    tg0, tg1, tg2, xn9 = _prep(xc, p9, wt0.T, wt1.T, wt2.T)
    mg0, mg1, mg2a, mg2b = _sc_messages(tg0, tg1, tg2, act, idx_i, idx_j, n)
    oc = _tail(mg0, mg1, mg2a, mg2b, tg0, tg1, tg2, xn9, p9.T,
               wt3.T, wt4.T, wt5.T, n)
    return oc.reshape(n, NC, 3, 3)
